# baseline (speedup 1.0000x reference)
import functools

import jax
import jax.numpy as jnp
from jax import lax
from jax.experimental import pallas as pl
from jax.experimental.pallas import tpu as pltpu
from jax.experimental.pallas import tpu_sc as plsc

N_NODES = 10000
D_IN = 128
D_HID = 256
D_OUT = 128

NC = 2
NS = 16
NW = NC * NS
CH = 128

N_ACC = 10112
RPT = N_ACC // NS
ROW_BLK = 2528

_MESH = plsc.VectorSubcoreMesh(core_axis_name="c", subcore_axis_name="s")


def _zero_rows(buf, nrows, width):
    zero16 = jnp.zeros((16,), jnp.float32)

    @pl.loop(0, nrows)
    def _(r):
        @pl.loop(0, width, step=16)
        def _(c):
            buf[r, pl.ds(c, 16)] = zero16


def _zero_acc_slice(zbuf, acc, base, width_rows):
    nfull = RPT // width_rows
    rem = RPT - nfull * width_rows

    @pl.loop(0, nfull)
    def _(k):
        pltpu.sync_copy(zbuf, acc.at[pl.ds(base + k * width_rows, width_rows)])

    if rem:
        pltpu.sync_copy(zbuf.at[pl.ds(0, rem)],
                        acc.at[pl.ds(base + nfull * width_rows, rem)])


FAST_CORE = 1
A_CHUNKS = 128
B_CHUNKS = 32
PASS_CH = 32


def _make_agg():
    scratch = [
        pltpu.VMEM((PASS_CH, CH), jnp.int32),
        pltpu.VMEM((PASS_CH, CH), jnp.int32),
        pltpu.VMEM((CH, D_IN), jnp.float32),
        pltpu.VMEM((CH, D_IN), jnp.float32),
        pltpu.VMEM_SHARED((N_ACC, D_IN), jnp.float32),
        pltpu.SemaphoreType.DMA,
        pltpu.SemaphoreType.DMA,
    ]

    def body(x_hbm, src_hbm, dst_hbm, sum_out,
             src_v, dst_v, rows0, rows1, acc, sem0, sem1):
        cid = lax.axis_index("c")
        sid = lax.axis_index("s")
        base = sid * RPT

        _zero_rows(rows0, CH, D_IN)
        _zero_acc_slice(rows0, acc, base, CH)
        plsc.subcore_barrier()

        def issue(c, rows, sem):
            pltpu.async_copy(x_hbm.at[src_v.at[c]], rows, sem)

        def drain(rows, sem):
            pltpu.make_async_copy(x_hbm.at[pl.ds(0, CH)], rows, sem).wait()

        def scat(c, rows):
            pltpu.sync_copy(rows, acc.at[dst_v.at[c]], add=True)

        def run_passes(npass, cpp, tile_base):
            @pl.loop(0, npass)
            def _(p):
                row0 = tile_base + p * cpp
                pltpu.sync_copy(src_hbm.at[pl.ds(row0, cpp)],
                                src_v.at[pl.ds(0, cpp)])
                pltpu.sync_copy(dst_hbm.at[pl.ds(row0, cpp)],
                                dst_v.at[pl.ds(0, cpp)])

                issue(0, rows0, sem0)
                issue(1, rows1, sem1)

                @pl.loop(0, cpp - 2, step=2)
                def _(c):
                    drain(rows0, sem0)
                    scat(c, rows0)
                    issue(c + 2, rows0, sem0)
                    drain(rows1, sem1)
                    scat(c + 1, rows1)
                    issue(c + 3, rows1, sem1)

                drain(rows0, sem0)
                scat(cpp - 2, rows0)
                drain(rows1, sem1)
                scat(cpp - 1, rows1)

        @pl.when(cid == FAST_CORE)
        def _():
            run_passes(A_CHUNKS // PASS_CH, PASS_CH, sid * A_CHUNKS)

        @pl.when(cid != FAST_CORE)
        def _():
            run_passes(max(1, B_CHUNKS // PASS_CH), min(B_CHUNKS, PASS_CH),
                       NS * A_CHUNKS + sid * B_CHUNKS)

        plsc.subcore_barrier()
        pltpu.sync_copy(acc.at[pl.ds(base, RPT)],
                        sum_out.at[cid, pl.ds(base, RPT)])

    return pl.kernel(
        body,
        out_type=jax.ShapeDtypeStruct((NC, N_ACC, D_IN), jnp.float32),
        mesh=_MESH,
        scratch_types=scratch,
    )


def _make_cnt(cpt: int):
    scratch = [
        pltpu.VMEM((cpt, CH), jnp.int32),
        pltpu.VMEM((CH, D_IN), jnp.float32),
        pltpu.VMEM_SHARED((N_ACC, D_IN), jnp.float32),
        pltpu.SemaphoreType.DMA,
    ]

    def body(dst_hbm, cnt_out, dst_v, ones_v, cacc, sem):
        cid = lax.axis_index("c")
        sid = lax.axis_index("s")
        wid = sid * NC + cid
        base = sid * RPT

        pltpu.sync_copy(dst_hbm.at[pl.ds(wid * cpt, cpt)], dst_v)

        _zero_rows(ones_v, CH, D_IN)
        _zero_acc_slice(ones_v, cacc, base, CH)

        one16 = jnp.ones((16,), jnp.float32)

        @pl.loop(0, CH)
        def _(r):
            @pl.loop(0, D_IN, step=16)
            def _(c):
                ones_v[r, pl.ds(c, 16)] = one16

        plsc.subcore_barrier()

        grp = 8

        @pl.loop(0, cpt, step=grp)
        def _(c0):
            for k in range(grp):
                pltpu.async_copy(ones_v, cacc.at[dst_v.at[c0 + k]], sem,
                                 add=True)
            for k in range(grp):
                pltpu.make_async_copy(ones_v, cacc.at[dst_v.at[c0]],
                                      sem).wait()

        plsc.subcore_barrier()
        pltpu.sync_copy(cacc.at[pl.ds(base, RPT)],
                        cnt_out.at[cid, pl.ds(base, RPT)])

    return pl.kernel(
        body,
        out_type=jax.ShapeDtypeStruct((NC, N_ACC, D_IN), jnp.float32),
        mesh=_MESH,
        scratch_types=scratch,
    )


def _tc1_body(s3, c3, xb, w1l, w1r, w2l, w2r, b1, b2, hl_out, r2_out):
    cnt = c3[0, :, 0:1] + c3[1, :, 0:1]
    inv = 1.0 / jnp.maximum(cnt, 1.0)
    mean = (s3[0] + s3[1]) * inv
    h = jnp.dot(mean, w1l[...], preferred_element_type=jnp.float32)
    h = h + jnp.dot(xb[...], w1r[...], preferred_element_type=jnp.float32)
    h = jnp.maximum(h + b1[...], 0.0)
    hl_out[...] = jnp.dot(h, w2l[...], preferred_element_type=jnp.float32)
    r2_out[...] = jnp.dot(h, w2r[...], preferred_element_type=jnp.float32) + b2[...]


def _tc2_body(s3, c3, r2, out):
    cnt = c3[0, :, 0:1] + c3[1, :, 0:1]
    inv = 1.0 / jnp.maximum(cnt, 1.0)
    out[...] = (s3[0] + s3[1]) * inv + r2[...]


def _row_spec(width):
    return pl.BlockSpec((2, ROW_BLK, width), lambda i: (0, i, 0))


def _full_spec(shape):
    return pl.BlockSpec(shape, lambda i: tuple(0 for _ in shape))


def kernel(x, edge_index, W1l, b1, W1r, W2l, b2, W2r):
    src = edge_index[0].astype(jnp.int32)
    dst = edge_index[1].astype(jnp.int32)
    e = src.shape[0]
    ct = NS * (A_CHUNKS + B_CHUNKS)
    e_pad = ct * CH
    src_p = jnp.concatenate(
        [src, jnp.zeros((e_pad - e,), jnp.int32)]).reshape(ct, CH)
    dst_p = jnp.concatenate(
        [dst, jnp.full((e_pad - e,), N_NODES, jnp.int32)]).reshape(ct, CH)

    cnt1 = _make_cnt(ct // NW)(dst_p)
    sum1 = _make_agg()(x, src_p, dst_p)

    grid = N_ACC // ROW_BLK
    hl, r2 = pl.pallas_call(
        _tc1_body,
        grid=(grid,),
        in_specs=[
            _row_spec(D_IN), _row_spec(D_IN),
            pl.BlockSpec((ROW_BLK, D_IN), lambda i: (i, 0)),
            _full_spec((D_IN, D_HID)), _full_spec((D_IN, D_HID)),
            _full_spec((D_HID, D_OUT)), _full_spec((D_HID, D_OUT)),
            _full_spec((1, D_HID)), _full_spec((1, D_OUT)),
        ],
        out_specs=[pl.BlockSpec((ROW_BLK, D_IN), lambda i: (i, 0)),
                   pl.BlockSpec((ROW_BLK, D_OUT), lambda i: (i, 0))],
        out_shape=[jax.ShapeDtypeStruct((N_NODES, D_OUT), jnp.float32),
                   jax.ShapeDtypeStruct((N_NODES, D_OUT), jnp.float32)],
    )(sum1, cnt1, x, W1l.T, W1r.T, W2l.T, W2r.T,
      b1.reshape(1, D_HID), b2.reshape(1, D_OUT))

    sum2 = _make_agg()(hl, src_p, dst_p)

    out = pl.pallas_call(
        _tc2_body,
        grid=(grid,),
        in_specs=[_row_spec(D_OUT), _row_spec(D_IN),
                  pl.BlockSpec((ROW_BLK, D_OUT), lambda i: (i, 0))],
        out_specs=pl.BlockSpec((ROW_BLK, D_OUT), lambda i: (i, 0)),
        out_shape=jax.ShapeDtypeStruct((N_NODES, D_OUT), jnp.float32),
    )(sum2, cnt1, r2)

    return out


# speedup vs baseline: 1.0645x; 1.0645x over previous
import functools

import jax
import jax.numpy as jnp
from jax import lax
from jax.experimental import pallas as pl
from jax.experimental.pallas import tpu as pltpu
from jax.experimental.pallas import tpu_sc as plsc

N_NODES = 10000
D_IN = 128
D_HID = 256
D_OUT = 128

NC = 2
NS = 16
NW = NC * NS
CH = 128

N_ACC = 10112
RPT = N_ACC // NS
ROW_BLK = 2528

_MESH = plsc.VectorSubcoreMesh(core_axis_name="c", subcore_axis_name="s")


def _zero_rows(buf, nrows, width):
    zero16 = jnp.zeros((16,), jnp.float32)

    @pl.loop(0, nrows)
    def _(r):
        @pl.loop(0, width, step=16)
        def _(c):
            buf[r, pl.ds(c, 16)] = zero16


def _zero_acc_slice(zbuf, acc, base, width_rows):
    nfull = RPT // width_rows
    rem = RPT - nfull * width_rows

    @pl.loop(0, nfull)
    def _(k):
        pltpu.sync_copy(zbuf, acc.at[pl.ds(base + k * width_rows, width_rows)])

    if rem:
        pltpu.sync_copy(zbuf.at[pl.ds(0, rem)],
                        acc.at[pl.ds(base + nfull * width_rows, rem)])


FAST_CORE = 1
A_CHUNKS = 128
B_CHUNKS = 32
PASS_CH = 32


def _make_agg():
    scratch = [
        pltpu.VMEM((PASS_CH, CH), jnp.int32),
        pltpu.VMEM((PASS_CH, CH), jnp.int32),
        pltpu.VMEM((CH, D_IN), jnp.float32),
        pltpu.VMEM((CH, D_IN), jnp.float32),
        pltpu.VMEM_SHARED((N_ACC, D_IN), jnp.float32),
        pltpu.SemaphoreType.DMA,
        pltpu.SemaphoreType.DMA,
    ]

    def body(x_hbm, src_hbm, dst_hbm, sum_out,
             src_v, dst_v, rows0, rows1, acc, sem0, sem1):
        cid = lax.axis_index("c")
        sid = lax.axis_index("s")
        base = sid * RPT

        _zero_rows(rows0, CH, D_IN)
        _zero_acc_slice(rows0, acc, base, CH)
        plsc.subcore_barrier()

        def issue(c, rows, sem):
            pltpu.async_copy(x_hbm.at[src_v.at[c]], rows, sem)

        def drain(rows, sem):
            pltpu.make_async_copy(x_hbm.at[pl.ds(0, CH)], rows, sem).wait()

        def scat(c, rows):
            pltpu.sync_copy(rows, acc.at[dst_v.at[c]], add=True)

        def run_passes(npass, cpp, tile_base):
            @pl.loop(0, npass)
            def _(p):
                row0 = tile_base + p * cpp
                pltpu.sync_copy(src_hbm.at[pl.ds(row0, cpp)],
                                src_v.at[pl.ds(0, cpp)])
                pltpu.sync_copy(dst_hbm.at[pl.ds(row0, cpp)],
                                dst_v.at[pl.ds(0, cpp)])

                issue(0, rows0, sem0)
                issue(1, rows1, sem1)

                @pl.loop(0, cpp - 2, step=2)
                def _(c):
                    drain(rows0, sem0)
                    scat(c, rows0)
                    issue(c + 2, rows0, sem0)
                    drain(rows1, sem1)
                    scat(c + 1, rows1)
                    issue(c + 3, rows1, sem1)

                drain(rows0, sem0)
                scat(cpp - 2, rows0)
                drain(rows1, sem1)
                scat(cpp - 1, rows1)

        @pl.when(cid == FAST_CORE)
        def _():
            run_passes(A_CHUNKS // PASS_CH, PASS_CH, sid * A_CHUNKS)

        @pl.when(cid != FAST_CORE)
        def _():
            run_passes(max(1, B_CHUNKS // PASS_CH), min(B_CHUNKS, PASS_CH),
                       NS * A_CHUNKS + sid * B_CHUNKS)

        plsc.subcore_barrier()
        pltpu.sync_copy(acc.at[pl.ds(base, RPT)],
                        sum_out.at[cid, pl.ds(base, RPT)])

    return pl.kernel(
        body,
        out_type=jax.ShapeDtypeStruct((NC, N_ACC, D_IN), jnp.float32),
        mesh=_MESH,
        scratch_types=scratch,
    )


def _make_cnt(cpt: int):
    scratch = [
        pltpu.VMEM((cpt, CH), jnp.int32),
        pltpu.VMEM((CH, D_IN), jnp.float32),
        pltpu.VMEM_SHARED((N_ACC, D_IN), jnp.float32),
        pltpu.SemaphoreType.DMA,
    ]

    def body(dst_hbm, cnt_out, dst_v, ones_v, cacc, sem):
        cid = lax.axis_index("c")
        sid = lax.axis_index("s")
        wid = sid * NC + cid
        base = sid * RPT

        pltpu.sync_copy(dst_hbm.at[pl.ds(wid * cpt, cpt)], dst_v)

        _zero_rows(ones_v, CH, D_IN)
        _zero_acc_slice(ones_v, cacc, base, CH)

        one16 = jnp.ones((16,), jnp.float32)

        @pl.loop(0, CH)
        def _(r):
            @pl.loop(0, D_IN, step=16)
            def _(c):
                ones_v[r, pl.ds(c, 16)] = one16

        plsc.subcore_barrier()

        grp = 8

        @pl.loop(0, cpt, step=grp)
        def _(c0):
            for k in range(grp):
                pltpu.async_copy(ones_v, cacc.at[dst_v.at[c0 + k]], sem,
                                 add=True)
            for k in range(grp):
                pltpu.make_async_copy(ones_v, cacc.at[dst_v.at[c0]],
                                      sem).wait()

        plsc.subcore_barrier()
        pltpu.sync_copy(cacc.at[pl.ds(base, RPT)],
                        cnt_out.at[cid, pl.ds(base, RPT)])

    return pl.kernel(
        body,
        out_type=jax.ShapeDtypeStruct((NC, N_ACC, D_IN), jnp.float32),
        mesh=_MESH,
        scratch_types=scratch,
    )


def _tc1_body(s3, c3, xb, w1l, w1r, w2l, w2r, b1, b2, hl_out, r2_out):
    cnt = c3[0, :, 0:1] + c3[1, :, 0:1]
    inv = 1.0 / jnp.maximum(cnt, 1.0)
    mean = (s3[0] + s3[1]) * inv
    h = jnp.dot(mean, w1l[...], preferred_element_type=jnp.float32)
    h = h + jnp.dot(xb[...], w1r[...], preferred_element_type=jnp.float32)
    h = jnp.maximum(h + b1[...], 0.0)
    hl_out[...] = jnp.dot(h, w2l[...], preferred_element_type=jnp.float32)
    r2_out[...] = jnp.dot(h, w2r[...], preferred_element_type=jnp.float32) + b2[...]


def _tc2_body(s3, c3, r2, out):
    cnt = c3[0, :, 0:1] + c3[1, :, 0:1]
    inv = 1.0 / jnp.maximum(cnt, 1.0)
    out[...] = (s3[0] + s3[1]) * inv + r2[...]


def _row_spec(width):
    return pl.BlockSpec((2, ROW_BLK, width), lambda i: (0, i, 0))


def _full_spec(shape):
    return pl.BlockSpec(shape, lambda i: tuple(0 for _ in shape))


def kernel(x, edge_index, W1l, b1, W1r, W2l, b2, W2r):
    src = edge_index[0].astype(jnp.int32)
    dst = edge_index[1].astype(jnp.int32)
    e = src.shape[0]
    ct = NS * (A_CHUNKS + B_CHUNKS)
    e_pad = ct * CH
    src_p = jnp.concatenate(
        [src, jnp.zeros((e_pad - e,), jnp.int32)]).reshape(ct, CH)
    dst_p = jnp.concatenate(
        [dst, jnp.full((e_pad - e,), N_NODES, jnp.int32)]).reshape(ct, CH)

    x_pad = jnp.pad(x, ((0, N_ACC - N_NODES), (0, 0)))

    cnt1 = _make_cnt(ct // NW)(dst_p)
    sum1 = _make_agg()(x, src_p, dst_p)

    grid = N_ACC // ROW_BLK
    hl, r2 = pl.pallas_call(
        _tc1_body,
        grid=(grid,),
        in_specs=[
            _row_spec(D_IN), _row_spec(D_IN),
            pl.BlockSpec((ROW_BLK, D_IN), lambda i: (i, 0)),
            _full_spec((D_IN, D_HID)), _full_spec((D_IN, D_HID)),
            _full_spec((D_HID, D_OUT)), _full_spec((D_HID, D_OUT)),
            _full_spec((1, D_HID)), _full_spec((1, D_OUT)),
        ],
        out_specs=[pl.BlockSpec((ROW_BLK, D_IN), lambda i: (i, 0)),
                   pl.BlockSpec((ROW_BLK, D_OUT), lambda i: (i, 0))],
        out_shape=[jax.ShapeDtypeStruct((N_ACC, D_OUT), jnp.float32),
                   jax.ShapeDtypeStruct((N_ACC, D_OUT), jnp.float32)],
    )(sum1, cnt1, x_pad, W1l.T, W1r.T, W2l.T, W2r.T,
      b1.reshape(1, D_HID), b2.reshape(1, D_OUT))

    sum2 = _make_agg()(hl, src_p, dst_p)

    out = pl.pallas_call(
        _tc2_body,
        grid=(grid,),
        in_specs=[_row_spec(D_OUT), _row_spec(D_IN),
                  pl.BlockSpec((ROW_BLK, D_OUT), lambda i: (i, 0))],
        out_specs=pl.BlockSpec((ROW_BLK, D_OUT), lambda i: (i, 0)),
        out_shape=jax.ShapeDtypeStruct((N_ACC, D_OUT), jnp.float32),
    )(sum2, cnt1, r2)

    return out[:N_NODES]
